# baseline (speedup 1.0000x reference)
import functools

import jax
import jax.numpy as jnp
from jax import lax
from jax.experimental import pallas as pl
from jax.experimental.pallas import tpu as pltpu
from jax.experimental.pallas import tpu_sc as plsc

_MARGIN = 1.0
_LANES = 16


@functools.lru_cache(maxsize=None)
def _build_sc_loss(num_nodes, num_rel, dim, batch, num_neg):
    info = plsc.get_sparse_core_info()
    nc, ns = info.num_cores, info.num_subcores
    nw = nc * ns
    assert dim % _LANES == 0
    assert batch % (nw * _LANES) == 0
    per_w = batch // nw
    p = _LANES
    n_chunks = per_w // p
    assert n_chunks % 2 == 0
    rows = p * (1 + num_neg)
    assert rows <= 128
    assert (p * num_neg) % _LANES == 0
    pwords = 3 * p
    nwords = 3 * p * num_neg
    dchunks = dim // _LANES

    mesh = plsc.VectorSubcoreMesh(core_axis_name="c", subcore_axis_name="s")

    @functools.partial(
        pl.kernel,
        out_type=jax.ShapeDtypeStruct((nw * _LANES,), jnp.float32),
        mesh=mesh,
        compiler_params=pltpu.CompilerParams(needs_layout_passes=False),
        scratch_types=[
            pltpu.VMEM((n_chunks * pwords,), jnp.int32),
            pltpu.VMEM((n_chunks * nwords,), jnp.int32),
            pltpu.VMEM((2, rows), jnp.int32),
            pltpu.VMEM((2, rows), jnp.int32),
            pltpu.VMEM((2, rows), jnp.int32),
            pltpu.VMEM((2, rows, dim), jnp.float32),
            pltpu.VMEM((2, rows, dim), jnp.float32),
            pltpu.VMEM((2, rows, dim), jnp.float32),
            pltpu.VMEM((_LANES,), jnp.float32),
            pltpu.SemaphoreType.DMA,
            pltpu.SemaphoreType.DMA,
        ],
    )
    def sc_loss(h_hbm, t_hbm, r_hbm, ps_hbm, ns_hbm, out_hbm,
                praw, nraw, ihb, irb, itb, hb, rb, tb, ob, sem0, sem1):
        wid = lax.axis_index("s") * nc + lax.axis_index("c")
        lane = lax.iota(jnp.int32, _LANES)
        sems = (sem0, sem1)

        preloads = [
            pltpu.make_async_copy(
                ps_hbm.at[pl.ds(wid * n_chunks * pwords, n_chunks * pwords)],
                praw, sem0),
            pltpu.make_async_copy(
                ns_hbm.at[pl.ds(wid * n_chunks * nwords, n_chunks * nwords)],
                nraw, sem0),
        ]
        for cp in preloads:
            cp.start()
        for cp in preloads:
            cp.wait()

        lane3 = lane * 3

        def build_idx(c, b):
            pbase = c * pwords
            nbase = c * nwords
            for col, dst in ((0, ihb), (1, irb), (2, itb)):
                v = plsc.load_gather(praw, [pbase + lane3 + col])
                dst[b, pl.ds(0, _LANES)] = v
                for g in range(num_neg):
                    v = plsc.load_gather(
                        nraw, [nbase + 3 * _LANES * g + lane3 + col])
                    dst[b, pl.ds(p + _LANES * g, _LANES)] = v

        def copies(b):
            return (
                pltpu.make_async_copy(h_hbm.at[ihb.at[b]], hb.at[b], sems[b]),
                pltpu.make_async_copy(r_hbm.at[irb.at[b]], rb.at[b], sems[b]),
                pltpu.make_async_copy(t_hbm.at[itb.at[b]], tb.at[b], sems[b]),
            )

        def fire(b):
            for cp in copies(b):
                cp.start()

        def wait(b):
            for cp in copies(b):
                cp.wait()

        def triple_score(b, row):
            acc = (hb[b, row, pl.ds(0, _LANES)]
                   * rb[b, row, pl.ds(0, _LANES)]
                   * tb[b, row, pl.ds(0, _LANES)])
            for cc in range(1, dchunks):
                acc = acc + (hb[b, row, pl.ds(cc * _LANES, _LANES)]
                             * rb[b, row, pl.ds(cc * _LANES, _LANES)]
                             * tb[b, row, pl.ds(cc * _LANES, _LANES)])
            return jnp.sum(acc)

        def compute(b, loss_acc):
            def pscore(i, acc_vec):
                s = triple_score(b, i)
                return jnp.where(lane == i, s, acc_vec)

            pos_s = lax.fori_loop(0, p, pscore,
                                  jnp.zeros((_LANES,), jnp.float32))
            la = loss_acc
            for n in range(num_neg):
                def nscore(i, acc_vec, _n=n):
                    s = triple_score(b, p + i * num_neg + _n)
                    return jnp.where(lane == i, s, acc_vec)

                neg_s = lax.fori_loop(0, p, nscore,
                                      jnp.zeros((_LANES,), jnp.float32))
                la = la + jnp.maximum(_MARGIN - pos_s + neg_s, 0.0)
            return la

        build_idx(0, 0)
        fire(0)

        def outer(c2, loss_acc):
            la = loss_acc
            for b in range(2):
                c = c2 * 2 + b

                @pl.when(c + 1 < n_chunks)
                def _():
                    build_idx(c + 1, 1 - b)
                    fire(1 - b)

                wait(b)
                la = compute(b, la)
            return la

        loss = lax.fori_loop(0, n_chunks // 2, outer,
                             jnp.zeros((_LANES,), jnp.float32))
        ob[...] = loss
        pltpu.sync_copy(ob, out_hbm.at[pl.ds(wid * _LANES, _LANES)])

    return sc_loss


def kernel(head_embeddings, tail_embeddings, relation_embeddings,
           positive_samples, negative_samples):
    batch, num_neg = negative_samples.shape[:2]
    num_nodes, dim = head_embeddings.shape
    num_rel = relation_embeddings.shape[0]

    fn = _build_sc_loss(num_nodes, num_rel, dim, batch, num_neg)
    partials = fn(head_embeddings, tail_embeddings, relation_embeddings,
                  positive_samples.reshape(-1), negative_samples.reshape(-1))
    return jnp.sum(partials) / (batch * num_neg)


# speedup vs baseline: 1.2891x; 1.2891x over previous
import functools

import jax
import jax.numpy as jnp
from jax import lax
from jax.experimental import pallas as pl
from jax.experimental.pallas import tpu as pltpu
from jax.experimental.pallas import tpu_sc as plsc

_MARGIN = 1.0
_LANES = 16


@functools.lru_cache(maxsize=None)
def _build_sc_loss(num_nodes, num_rel, dim, batch, num_neg):
    info = plsc.get_sparse_core_info()
    nc, ns = info.num_cores, info.num_subcores
    nw = nc * ns
    assert dim % _LANES == 0
    assert batch % (nw * _LANES) == 0
    per_w = batch // nw
    p = _LANES
    n_chunks = per_w // p
    rows = p * (1 + num_neg)
    assert rows <= 128
    idx_per_w = n_chunks * rows
    dchunks = dim // _LANES

    mesh = plsc.VectorSubcoreMesh(core_axis_name="c", subcore_axis_name="s")

    @functools.partial(
        pl.kernel,
        out_type=jax.ShapeDtypeStruct((nw * _LANES,), jnp.float32),
        mesh=mesh,
        compiler_params=pltpu.CompilerParams(needs_layout_passes=False),
        scratch_types=[
            pltpu.VMEM((idx_per_w,), jnp.int32),
            pltpu.VMEM((idx_per_w,), jnp.int32),
            pltpu.VMEM((idx_per_w,), jnp.int32),
            pltpu.VMEM((2, rows, dim), jnp.float32),
            pltpu.VMEM((2, rows, dim), jnp.float32),
            pltpu.VMEM((2, rows, dim), jnp.float32),
            pltpu.VMEM((_LANES,), jnp.float32),
            pltpu.SemaphoreType.DMA,
            pltpu.SemaphoreType.DMA,
        ],
    )
    def sc_loss(h_hbm, t_hbm, r_hbm, hi_hbm, ri_hbm, ti_hbm, out_hbm,
                ih, ir, it, hb, rb, tb, ob, sem0, sem1):
        wid = lax.axis_index("s") * nc + lax.axis_index("c")
        idx_base = wid * idx_per_w
        lane = lax.iota(jnp.int32, _LANES)
        sems = (sem0, sem1)

        preloads = [
            pltpu.make_async_copy(
                src.at[pl.ds(idx_base, idx_per_w)], dst, sem0)
            for src, dst in ((hi_hbm, ih), (ri_hbm, ir), (ti_hbm, it))
        ]
        for cp in preloads:
            cp.start()
        for cp in preloads:
            cp.wait()

        def copies(c, b):
            off = c * rows
            return (
                pltpu.make_async_copy(
                    h_hbm.at[ih.at[pl.ds(off, rows)]], hb.at[b], sems[b]),
                pltpu.make_async_copy(
                    r_hbm.at[ir.at[pl.ds(off, rows)]], rb.at[b], sems[b]),
                pltpu.make_async_copy(
                    t_hbm.at[it.at[pl.ds(off, rows)]], tb.at[b], sems[b]),
            )

        def fire(c, b):
            for cp in copies(c, b):
                cp.start()

        def wait(c, b):
            for cp in copies(c, b):
                cp.wait()

        def triple_score(b, row):
            acc = (hb[b, row, pl.ds(0, _LANES)]
                   * rb[b, row, pl.ds(0, _LANES)]
                   * tb[b, row, pl.ds(0, _LANES)])
            for cc in range(1, dchunks):
                acc = acc + (hb[b, row, pl.ds(cc * _LANES, _LANES)]
                             * rb[b, row, pl.ds(cc * _LANES, _LANES)]
                             * tb[b, row, pl.ds(cc * _LANES, _LANES)])
            return jnp.sum(acc)

        def compute(b, loss_acc):
            def pscore(i, acc_vec):
                s = triple_score(b, i)
                return jnp.where(lane == i, s, acc_vec)

            pos_s = lax.fori_loop(0, p, pscore,
                                  jnp.zeros((_LANES,), jnp.float32))
            la = loss_acc
            for n in range(num_neg):
                def nscore(i, acc_vec, _n=n):
                    s = triple_score(b, p + i * num_neg + _n)
                    return jnp.where(lane == i, s, acc_vec)

                neg_s = lax.fori_loop(0, p, nscore,
                                      jnp.zeros((_LANES,), jnp.float32))
                la = la + jnp.maximum(_MARGIN - pos_s + neg_s, 0.0)
            return la

        fire(0, 0)

        def outer(c2, loss_acc):
            la = loss_acc
            for b in range(2):
                c = c2 * 2 + b

                @pl.when(c + 1 < n_chunks)
                def _():
                    fire(c + 1, 1 - b)

                wait(c, b)
                la = compute(b, la)
            return la

        loss = lax.fori_loop(0, n_chunks // 2, outer,
                             jnp.zeros((_LANES,), jnp.float32))
        ob[...] = loss
        pltpu.sync_copy(ob, out_hbm.at[pl.ds(wid * _LANES, _LANES)])

    return sc_loss


def kernel(head_embeddings, tail_embeddings, relation_embeddings,
           positive_samples, negative_samples):
    batch, num_neg = negative_samples.shape[:2]
    num_nodes, dim = head_embeddings.shape
    num_rel = relation_embeddings.shape[0]
    p = _LANES
    n_chunks_total = batch // p

    pT = positive_samples.T
    nT = jnp.transpose(negative_samples, (2, 0, 1))
    cols = []
    for j in range(3):
        cols.append(jnp.concatenate(
            [pT[j].reshape(n_chunks_total, p),
             nT[j].reshape(n_chunks_total, p * num_neg)], axis=1).reshape(-1))
    hidx, ridx, tidx = cols

    fn = _build_sc_loss(num_nodes, num_rel, dim, batch, num_neg)
    partials = fn(head_embeddings, tail_embeddings, relation_embeddings,
                  hidx, ridx, tidx)
    return jnp.sum(partials) / (batch * num_neg)
